# baseline (speedup 1.0000x reference)
import jax
import jax.numpy as jnp
from jax import lax
from jax.experimental import pallas as pl
from jax.experimental.pallas import tpu as pltpu
from jax.experimental.pallas import tpu_sc as plsc

_VOCAB = 100000
_D = 128
_B = 4096
_SEG = 200
_HID = 256
_NPRED = 1000

_NC, _NS = 2, 16
_NW = _NC * _NS
_RW = _B // _NW
_CH = 100
_CPR = _SEG // _CH
_NCHUNK = _RW * _CPR
_NB = 8
_RPI = _NB // _CPR

_PERM = tuple(32 * k + 2 * j + p for k in range(4) for p in (0, 1)
              for j in range(16))


def _pool_body(x_hbm, table_hbm, out_hbm, idx_v, bufs, stage, *sems):
    c = lax.axis_index("c")
    s = lax.axis_index("s")
    w = c * _NS + s
    pltpu.sync_copy(x_hbm.at[w], idx_v)

    def gather(cidx, slot):
        return pltpu.make_async_copy(
            table_hbm.at[idx_v.at[cidx]], bufs.at[slot], sems[slot])

    for b in range(_NB):
        gather(b, b).start()

    def accum_chunk(slot, accs):
        def body(j, a):
            out = list(a)
            for u in range(2):
                row = j * 2 + u
                for k in range(4):
                    wd = bufs[slot, row, pl.ds(k * 16, 16)]
                    lo = lax.bitcast_convert_type(
                        jnp.left_shift(wd, 16), jnp.float32)
                    hi = lax.bitcast_convert_type(wd, jnp.float32)
                    out[2 * k] = out[2 * k] + lo
                    out[2 * k + 1] = out[2 * k + 1] + hi
            return tuple(out)
        return lax.fori_loop(0, _CH // 2, body, accs)

    def row_group(i, issue_next):
        for rr in range(_RPI):
            r = _RPI * i + rr
            accs = tuple(jnp.zeros((16,), jnp.float32) for _ in range(8))
            for h in range(_CPR):
                slot = _CPR * rr + h
                cidx = _NB * i + slot
                gather(cidx, slot).wait()
                accs = accum_chunk(slot, accs)
                if issue_next:
                    gather(cidx + _NB, slot).start()
            for k in range(8):
                stage[r, pl.ds(k * 16, 16)] = accs[k]

    def loop_body(i, carry):
        row_group(i, True)
        return carry

    lax.fori_loop(0, _RW // _RPI - 1, loop_body, 0)
    row_group(_RW // _RPI - 1, False)

    pltpu.sync_copy(stage, out_hbm.at[pl.ds(w * _RW, _RW)])


_pool = pl.kernel(
    _pool_body,
    out_type=jax.ShapeDtypeStruct((_B, _D), jnp.float32),
    mesh=plsc.VectorSubcoreMesh(
        core_axis_name="c", subcore_axis_name="s",
        num_cores=_NC, num_subcores=_NS),
    scratch_types=[
        pltpu.VMEM((_NCHUNK, _CH), jnp.int32),
        pltpu.VMEM((_NB, _CH, _D // 2), jnp.int32),
        pltpu.VMEM((_RW, _D), jnp.float32),
    ] + [pltpu.SemaphoreType.DMA] * _NB,
    compiler_params=pltpu.CompilerParams(
        use_tc_tiling_on_sc=False, needs_layout_passes=False),
)

_BB = 512
_NPAD = 1024


def _mlp_body(s_ref, w1_ref, b1_ref, w2_ref, b2_ref, out_ref):
    sv = s_ref[...].astype(jnp.float32)
    h = jnp.dot(sv, w1_ref[...], preferred_element_type=jnp.float32)
    h = h + b1_ref[...]
    h = 1.0 / (1.0 + jnp.exp(-h))
    logits = jnp.dot(h, w2_ref[...], preferred_element_type=jnp.float32)
    logits = logits + b2_ref[...]
    m = jnp.max(logits, axis=1, keepdims=True)
    lse = jnp.log(jnp.sum(jnp.exp(logits - m), axis=1, keepdims=True)) + m
    out_ref[...] = logits - lse


_mlp = pl.pallas_call(
    _mlp_body,
    grid=(_B // _BB,),
    in_specs=[
        pl.BlockSpec((_BB, _D), lambda i: (i, 0)),
        pl.BlockSpec((_D, _HID), lambda i: (0, 0)),
        pl.BlockSpec((1, _HID), lambda i: (0, 0)),
        pl.BlockSpec((_HID, _NPAD), lambda i: (0, 0)),
        pl.BlockSpec((1, _NPAD), lambda i: (0, 0)),
    ],
    out_specs=pl.BlockSpec((_BB, _NPAD), lambda i: (i, 0)),
    out_shape=jax.ShapeDtypeStruct((_B, _NPAD), jnp.float32),
)


def kernel(x, table, W1, b1, W2, b2):
    xr = x.reshape(_NW, _NCHUNK, _CH)
    tb = lax.bitcast_convert_type(
        table.astype(jnp.bfloat16).reshape(_VOCAB, _D // 2, 2), jnp.int32)
    s = _pool(xr, tb)
    W2p = jnp.concatenate(
        [W2, jnp.zeros((_HID, _NPAD - _NPRED), W2.dtype)], axis=1)
    b2p = jnp.concatenate(
        [b2, jnp.full((_NPAD - _NPRED,), -1e30, b2.dtype)])
    out = _mlp(s, W1[jnp.array(_PERM)], b1.reshape(1, _HID), W2p,
               b2p.reshape(1, _NPAD))
    return out[:, :_NPRED]


# speedup vs baseline: 2.8697x; 2.8697x over previous
import jax
import jax.numpy as jnp
from jax import lax
from jax.experimental import pallas as pl
from jax.experimental.pallas import tpu as pltpu
from jax.experimental.pallas import tpu_sc as plsc

_VOCAB = 100000
_D = 128
_B = 4096
_SEG = 200
_HID = 256
_NPRED = 1000

_NC, _NS = 2, 16
_NW = _NC * _NS
_RW = _B // _NW
_CH = 100
_CPR = _SEG // _CH
_NCHUNK = _RW * _CPR
_NB = 8
_RPI = _NB // _CPR

_VW = _VOCAB // _NW
_PCH = 125
_NPCH = _VW // _PCH


def _pack_body(table_hbm, out_hbm, inb, outb, sem_i, sem_o):
    c = lax.axis_index("c")
    s = lax.axis_index("s")
    w = c * _NS + s
    base = w * _VW

    def load(ch, slot):
        return pltpu.make_async_copy(
            table_hbm.at[pl.ds(base + ch * _PCH, _PCH)], inb.at[slot], sem_i)

    def store(ch, slot):
        return pltpu.make_async_copy(
            outb.at[slot], out_hbm.at[pl.ds(base + ch * _PCH, _PCH)], sem_o)

    def compute(slot):
        def rbody(r, carry):
            for k in range(4):
                a = inb[slot, r, pl.ds(32 * k, 16)]
                bv = inb[slot, r, pl.ds(32 * k + 16, 16)]
                ai = lax.bitcast_convert_type(a, jnp.int32)
                bi = lax.bitcast_convert_type(bv, jnp.int32)
                ar = ai + 32767 + jnp.bitwise_and(jnp.right_shift(ai, 16), 1)
                br = bi + 32767 + jnp.bitwise_and(jnp.right_shift(bi, 16), 1)
                wd = jnp.bitwise_or(
                    jnp.bitwise_and(jnp.right_shift(ar, 16), 65535),
                    jnp.bitwise_and(br, -65536))
                outb[slot, r, pl.ds(16 * k, 16)] = wd
            return carry
        lax.fori_loop(0, _PCH, rbody, 0)

    load(0, 0).start()
    load(1, 1).start()
    for ch in range(_NPCH):
        slot = ch % 2
        load(ch, slot).wait()
        if ch >= 2:
            store(ch - 2, slot).wait()
        compute(slot)
        store(ch, slot).start()
        if ch + 2 < _NPCH:
            load(ch + 2, slot).start()
    store(_NPCH - 2, (_NPCH - 2) % 2).wait()
    store(_NPCH - 1, (_NPCH - 1) % 2).wait()


_pack = pl.kernel(
    _pack_body,
    out_type=jax.ShapeDtypeStruct((_VOCAB, _D // 2), jnp.int32),
    mesh=plsc.VectorSubcoreMesh(
        core_axis_name="c", subcore_axis_name="s",
        num_cores=_NC, num_subcores=_NS),
    scratch_types=[
        pltpu.VMEM((2, _PCH, _D), jnp.float32),
        pltpu.VMEM((2, _PCH, _D // 2), jnp.int32),
        pltpu.SemaphoreType.DMA,
        pltpu.SemaphoreType.DMA,
    ],
    compiler_params=pltpu.CompilerParams(
        use_tc_tiling_on_sc=False, needs_layout_passes=False),
)


def _pool_body(x_hbm, table_hbm, out_hbm, idx_v, bufs, stage, *sems):
    c = lax.axis_index("c")
    s = lax.axis_index("s")
    w = c * _NS + s
    pltpu.sync_copy(x_hbm.at[w], idx_v)

    def gather(cidx, slot):
        return pltpu.make_async_copy(
            table_hbm.at[idx_v.at[cidx]], bufs.at[slot], sems[slot])

    for b in range(_NB):
        gather(b, b).start()

    def accum_chunk(slot, accs):
        def body(j, a):
            out = list(a)
            for u in range(2):
                row = j * 2 + u
                for k in range(4):
                    wd = bufs[slot, row, pl.ds(k * 16, 16)]
                    lo = lax.bitcast_convert_type(
                        jnp.left_shift(wd, 16), jnp.float32)
                    hi = lax.bitcast_convert_type(wd, jnp.float32)
                    out[2 * k] = out[2 * k] + lo
                    out[2 * k + 1] = out[2 * k + 1] + hi
            return tuple(out)
        return lax.fori_loop(0, _CH // 2, body, accs)

    def row_group(i, issue_next):
        for rr in range(_RPI):
            r = _RPI * i + rr
            accs = tuple(jnp.zeros((16,), jnp.float32) for _ in range(8))
            for h in range(_CPR):
                slot = _CPR * rr + h
                cidx = _NB * i + slot
                gather(cidx, slot).wait()
                accs = accum_chunk(slot, accs)
                if issue_next:
                    gather(cidx + _NB, slot).start()
            for k in range(8):
                stage[r, pl.ds(k * 16, 16)] = accs[k]

    def loop_body(i, carry):
        row_group(i, True)
        return carry

    lax.fori_loop(0, _RW // _RPI - 1, loop_body, 0)
    row_group(_RW // _RPI - 1, False)

    pltpu.sync_copy(stage, out_hbm.at[pl.ds(w * _RW, _RW)])


_pool = pl.kernel(
    _pool_body,
    out_type=jax.ShapeDtypeStruct((_B, _D), jnp.float32),
    mesh=plsc.VectorSubcoreMesh(
        core_axis_name="c", subcore_axis_name="s",
        num_cores=_NC, num_subcores=_NS),
    scratch_types=[
        pltpu.VMEM((_NCHUNK, _CH), jnp.int32),
        pltpu.VMEM((_NB, _CH, _D // 2), jnp.int32),
        pltpu.VMEM((_RW, _D), jnp.float32),
    ] + [pltpu.SemaphoreType.DMA] * _NB,
    compiler_params=pltpu.CompilerParams(
        use_tc_tiling_on_sc=False, needs_layout_passes=False),
)

_BB = 512
_NPAD = 1024


def _mlp_body(s_ref, w1_ref, b1_ref, w2_ref, b2_ref, out_ref):
    sv = s_ref[...].astype(jnp.float32)
    h = jnp.dot(sv, w1_ref[...], preferred_element_type=jnp.float32)
    h = h + b1_ref[...]
    h = 1.0 / (1.0 + jnp.exp(-h))
    logits = jnp.dot(h, w2_ref[...], preferred_element_type=jnp.float32)
    logits = logits + b2_ref[...]
    m = jnp.max(logits, axis=1, keepdims=True)
    lse = jnp.log(jnp.sum(jnp.exp(logits - m), axis=1, keepdims=True)) + m
    out_ref[...] = logits - lse


_mlp = pl.pallas_call(
    _mlp_body,
    grid=(_B // _BB,),
    in_specs=[
        pl.BlockSpec((_BB, _D), lambda i: (i, 0)),
        pl.BlockSpec((_D, _HID), lambda i: (0, 0)),
        pl.BlockSpec((1, _HID), lambda i: (0, 0)),
        pl.BlockSpec((_HID, _NPAD), lambda i: (0, 0)),
        pl.BlockSpec((1, _NPAD), lambda i: (0, 0)),
    ],
    out_specs=pl.BlockSpec((_BB, _NPAD), lambda i: (i, 0)),
    out_shape=jax.ShapeDtypeStruct((_B, _NPAD), jnp.float32),
)


def kernel(x, table, W1, b1, W2, b2):
    xr = x.reshape(_NW, _NCHUNK, _CH)
    tb = _pack(table)
    s = _pool(xr, tb)
    W2p = jnp.concatenate(
        [W2, jnp.zeros((_HID, _NPAD - _NPRED), W2.dtype)], axis=1)
    b2p = jnp.concatenate(
        [b2, jnp.full((_NPAD - _NPRED,), -1e30, b2.dtype)])
    out = _mlp(s, W1, b1.reshape(1, _HID), W2p, b2p.reshape(1, _NPAD))
    return out[:, :_NPRED]


# speedup vs baseline: 3.3690x; 1.1740x over previous
import jax
import jax.numpy as jnp
from jax import lax
from jax.experimental import pallas as pl
from jax.experimental.pallas import tpu as pltpu
from jax.experimental.pallas import tpu_sc as plsc

_VOCAB = 100000
_D = 128
_B = 4096
_SEG = 200
_HID = 256
_NPRED = 1000

_NC, _NS = 2, 16
_NW = _NC * _NS
_RW = _B // _NW
_CH = 100
_CPR = _SEG // _CH
_NCHUNK = _RW * _CPR
_NB = 8
_RPI = _NB // _CPR

_VW = _VOCAB // _NW
_PCH = 125
_NPCH = _VW // _PCH


def _pack_body(table_hbm, out_hbm, inb, outb, sem_i, sem_o):
    c = lax.axis_index("c")
    s = lax.axis_index("s")
    w = c * _NS + s
    base = w * _VW

    def load(ch, slot):
        return pltpu.make_async_copy(
            table_hbm.at[pl.ds(base + ch * _PCH, _PCH)], inb.at[slot], sem_i)

    def store(ch, slot):
        return pltpu.make_async_copy(
            outb.at[slot], out_hbm.at[pl.ds(base + ch * _PCH, _PCH)], sem_o)

    def compute(slot):
        def rbody(r, carry):
            for k in range(4):
                a = inb[slot, r, pl.ds(32 * k, 16)]
                bv = inb[slot, r, pl.ds(32 * k + 16, 16)]
                ai = lax.bitcast_convert_type(a, jnp.int32)
                bi = lax.bitcast_convert_type(bv, jnp.int32)
                wd = jnp.bitwise_or(
                    jnp.bitwise_and(jnp.right_shift(ai, 16), 65535),
                    jnp.bitwise_and(bi, -65536))
                outb[slot, r, pl.ds(16 * k, 16)] = wd
            return carry
        lax.fori_loop(0, _PCH, rbody, 0)

    load(0, 0).start()
    load(1, 1).start()
    for ch in range(_NPCH):
        slot = ch % 2
        load(ch, slot).wait()
        if ch >= 2:
            store(ch - 2, slot).wait()
        compute(slot)
        store(ch, slot).start()
        if ch + 2 < _NPCH:
            load(ch + 2, slot).start()
    store(_NPCH - 2, (_NPCH - 2) % 2).wait()
    store(_NPCH - 1, (_NPCH - 1) % 2).wait()


_pack = pl.kernel(
    _pack_body,
    out_type=jax.ShapeDtypeStruct((_VOCAB, _D // 2), jnp.int32),
    mesh=plsc.VectorSubcoreMesh(
        core_axis_name="c", subcore_axis_name="s",
        num_cores=_NC, num_subcores=_NS),
    scratch_types=[
        pltpu.VMEM((2, _PCH, _D), jnp.float32),
        pltpu.VMEM((2, _PCH, _D // 2), jnp.int32),
        pltpu.SemaphoreType.DMA,
        pltpu.SemaphoreType.DMA,
    ],
    compiler_params=pltpu.CompilerParams(
        use_tc_tiling_on_sc=False, needs_layout_passes=False),
)


def _pool_body(x_hbm, table_hbm, out_hbm, idx_v, bufs, stage, *sems):
    c = lax.axis_index("c")
    s = lax.axis_index("s")
    w = c * _NS + s
    pltpu.sync_copy(x_hbm.at[w], idx_v)

    def gather(cidx, slot):
        return pltpu.make_async_copy(
            table_hbm.at[idx_v.at[cidx]], bufs.at[slot], sems[slot])

    for b in range(_NB):
        gather(b, b).start()

    def accum_chunk(slot, accs):
        def body(j, a):
            out = list(a)
            for u in range(2):
                row = j * 2 + u
                for k in range(4):
                    wd = bufs[slot, row, pl.ds(k * 16, 16)]
                    lo = lax.bitcast_convert_type(
                        jnp.left_shift(wd, 16), jnp.float32)
                    hi = lax.bitcast_convert_type(wd, jnp.float32)
                    out[2 * k] = out[2 * k] + lo
                    out[2 * k + 1] = out[2 * k + 1] + hi
            return tuple(out)
        return lax.fori_loop(0, _CH // 2, body, accs)

    def row_group(i, issue_next):
        for rr in range(_RPI):
            r = _RPI * i + rr
            accs = tuple(jnp.zeros((16,), jnp.float32) for _ in range(8))
            for h in range(_CPR):
                slot = _CPR * rr + h
                cidx = _NB * i + slot
                gather(cidx, slot).wait()
                accs = accum_chunk(slot, accs)
                if issue_next:
                    gather(cidx + _NB, slot).start()
            for k in range(8):
                stage[r, pl.ds(k * 16, 16)] = accs[k]

    def loop_body(i, carry):
        row_group(i, True)
        return carry

    lax.fori_loop(0, _RW // _RPI - 1, loop_body, 0)
    row_group(_RW // _RPI - 1, False)

    pltpu.sync_copy(stage, out_hbm.at[pl.ds(w * _RW, _RW)])


_pool = pl.kernel(
    _pool_body,
    out_type=jax.ShapeDtypeStruct((_B, _D), jnp.float32),
    mesh=plsc.VectorSubcoreMesh(
        core_axis_name="c", subcore_axis_name="s",
        num_cores=_NC, num_subcores=_NS),
    scratch_types=[
        pltpu.VMEM((_NCHUNK, _CH), jnp.int32),
        pltpu.VMEM((_NB, _CH, _D // 2), jnp.int32),
        pltpu.VMEM((_RW, _D), jnp.float32),
    ] + [pltpu.SemaphoreType.DMA] * _NB,
    compiler_params=pltpu.CompilerParams(
        use_tc_tiling_on_sc=False, needs_layout_passes=False),
)

_BB = 512
_NPAD = 1024


def _mlp_body(s_ref, w1_ref, b1_ref, w2_ref, b2_ref, out_ref):
    sv = s_ref[...].astype(jnp.float32)
    h = jnp.dot(sv, w1_ref[...], preferred_element_type=jnp.float32)
    h = h + b1_ref[...]
    h = 1.0 / (1.0 + jnp.exp(-h))
    logits = jnp.dot(h, w2_ref[...], preferred_element_type=jnp.float32)
    logits = logits + b2_ref[...]
    m = jnp.max(logits, axis=1, keepdims=True)
    lse = jnp.log(jnp.sum(jnp.exp(logits - m), axis=1, keepdims=True)) + m
    out_ref[...] = logits - lse


_mlp = pl.pallas_call(
    _mlp_body,
    grid=(_B // _BB,),
    in_specs=[
        pl.BlockSpec((_BB, _D), lambda i: (i, 0)),
        pl.BlockSpec((_D, _HID), lambda i: (0, 0)),
        pl.BlockSpec((1, _HID), lambda i: (0, 0)),
        pl.BlockSpec((_HID, _NPAD), lambda i: (0, 0)),
        pl.BlockSpec((1, _NPAD), lambda i: (0, 0)),
    ],
    out_specs=pl.BlockSpec((_BB, _NPAD), lambda i: (i, 0)),
    out_shape=jax.ShapeDtypeStruct((_B, _NPAD), jnp.float32),
)


def kernel(x, table, W1, b1, W2, b2):
    xr = x.reshape(_NW, _NCHUNK, _CH)
    tb = _pack(table)
    s = _pool(xr, tb)
    W2p = jnp.concatenate(
        [W2, jnp.zeros((_HID, _NPAD - _NPRED), W2.dtype)], axis=1)
    b2p = jnp.concatenate(
        [b2, jnp.full((_NPAD - _NPRED,), -1e30, b2.dtype)])
    out = _mlp(s, W1, b1.reshape(1, _HID), W2p, b2p.reshape(1, _NPAD))
    return out[:, :_NPRED]
